# baseline (speedup 1.0000x reference)
import functools

import jax
import jax.numpy as jnp
from jax import lax
from jax.experimental import pallas as pl
from jax.experimental.pallas import tpu as pltpu
from jax.experimental.pallas import tpu_sc as plsc

HW = 100
D = 128
NW = 32
CP_ROWS = 16
BASE_ROWS = 104
ROWS_PER_STEP = 4
T = ROWS_PER_STEP * HW


def _tables_body(ct_ref, pt_ref, wc_ref, wp_ref, b_ref, cp_ref, base_ref):
    cp_ref[...] = lax.dot_general(ct_ref[...], wc_ref[...],
                                  (((1,), (1,)), ((), ())),
                                  preferred_element_type=jnp.float32)
    pp = lax.dot_general(pt_ref[...], wp_ref[...], (((1,), (1,)), ((), ())),
                         preferred_element_type=jnp.float32)
    base_ref[...] = pp + b_ref[...]


def _make_tables(color_table, pos_table, W, b):
    q = color_table.shape[1]
    ct = jnp.pad(color_table, ((0, CP_ROWS - color_table.shape[0]), (0, 0)))
    pt = jnp.pad(pos_table, ((0, BASE_ROWS - pos_table.shape[0]), (0, 0)))
    return pl.pallas_call(
        _tables_body,
        out_shape=(jax.ShapeDtypeStruct((CP_ROWS, D), jnp.float32),
                   jax.ShapeDtypeStruct((BASE_ROWS, D), jnp.float32)),
    )(ct, pt, W[:, :q], W[:, q:], b.reshape(1, D))


def _sc_combine(cp, base, grid_flat):
    n = grid_flat.shape[0]
    n_ex = n // HW
    per_w = n // NW
    steps = per_w // T
    outer = steps // 2
    mesh = plsc.VectorSubcoreMesh(core_axis_name="c", subcore_axis_name="s")

    @functools.partial(
        pl.kernel,
        out_type=jax.ShapeDtypeStruct((n_ex, HW, D), jnp.float32),
        mesh=mesh,
        scratch_types=[
            pltpu.VMEM((CP_ROWS, D), jnp.float32),
            pltpu.VMEM((BASE_ROWS, D), jnp.float32),
            pltpu.VMEM((T + 16,), jnp.int32),
            pltpu.VMEM((T + 16,), jnp.int32),
            pltpu.VMEM((T, D), jnp.float32),
            pltpu.VMEM((T, D), jnp.float32),
            pltpu.SemaphoreType.DMA,
            pltpu.SemaphoreType.DMA,
        ],
    )
    def sc_fn(cp_hbm, base_hbm, grid_hbm, out_hbm,
              cp_v, base_v, gs0, gs1, r0, r1, ss0, ss1):
        wid = lax.axis_index("s") * 2 + lax.axis_index("c")
        base_row = wid * per_w
        gs, rows_v, ss = (gs0, gs1), (r0, r1), (ss0, ss1)

        pltpu.sync_copy(cp_hbm, cp_v)
        pltpu.sync_copy(base_hbm, base_v)

        def load_grid(step, buf):
            off = pl.multiple_of(base_row + step * T, T)
            pltpu.sync_copy(grid_hbm.at[pl.ds(off, T)], gs[buf].at[pl.ds(0, T)])

        def compute(buf):
            def p_body(p, carry):
                bch = [base_v[p, pl.ds(16 * m, 16)] for m in range(D // 16)]
                for e in range(ROWS_PER_STEP):
                    g = gs[buf][pl.ds(e * HW + p, 16)][0]
                    for m in range(D // 16):
                        rows_v[buf][e * HW + p, pl.ds(16 * m, 16)] = (
                            cp_v[g, pl.ds(16 * m, 16)] + bch[m])
                return carry
            lax.fori_loop(0, HW, p_body, 0)

        def fire_scatter(step, buf):
            ex = base_row // HW + step * ROWS_PER_STEP
            for e in range(ROWS_PER_STEP):
                pltpu.async_copy(rows_v[buf].at[pl.ds(e * HW, HW)],
                                 out_hbm.at[ex + e], ss[buf])

        def wait_scatter(buf):
            for e in range(ROWS_PER_STEP):
                pltpu.make_async_copy(rows_v[buf].at[pl.ds(e * HW, HW)],
                                      out_hbm.at[0], ss[buf]).wait()

        load_grid(0, 0)

        def outer_body(g, carry):
            s0 = 2 * g
            load_grid(s0 + 1, 1)
            @pl.when(g > 0)
            def _():
                wait_scatter(0)
            compute(0)
            fire_scatter(s0, 0)
            @pl.when(g < outer - 1)
            def _():
                load_grid(s0 + 2, 0)
            @pl.when(g > 0)
            def _():
                wait_scatter(1)
            compute(1)
            fire_scatter(s0 + 1, 1)
            return carry

        lax.fori_loop(0, outer, outer_body, 0)
        wait_scatter(0)
        wait_scatter(1)

    return sc_fn(cp, base, grid_flat)


def kernel(grid, color_table, pos_table, W, b):
    cp, base = _make_tables(color_table, pos_table, W, b)
    flat = grid.reshape(-1).astype(jnp.int32)
    return _sc_combine(cp, base, flat)


# speedup vs baseline: 3.0944x; 3.0944x over previous
import functools

import jax
import jax.numpy as jnp
from jax import lax
from jax.experimental import pallas as pl
from jax.experimental.pallas import tpu as pltpu
from jax.experimental.pallas import tpu_sc as plsc

HW = 100
D = 128
NW = 32
ROWS_PER_STEP = 4
T = ROWS_PER_STEP * HW
GCHUNK = 80
NG = T // GCHUNK


def _fused_body(ct_ref, pt_ref, wc_ref, wp_ref, b_ref, out_ref):
    cp = lax.dot_general(ct_ref[...], wc_ref[...], (((1,), (1,)), ((), ())),
                         preferred_element_type=jnp.float32)
    pp = lax.dot_general(pt_ref[...], wp_ref[...], (((1,), (1,)), ((), ())),
                         preferred_element_type=jnp.float32)
    pp = pp + b_ref[...]
    acc = cp[:, None, :] + pp[None, :, :]
    out_ref[...] = acc.reshape(10 * HW, D)


def _make_fused(color_table, pos_table, W, b):
    q = color_table.shape[1]
    return pl.pallas_call(
        _fused_body,
        out_shape=jax.ShapeDtypeStruct((10 * HW, D), jnp.float32),
    )(color_table, pos_table, W[:, :q], W[:, q:], b.reshape(1, D))


def _sc_gather(fused, grid_flat):
    n = grid_flat.shape[0]
    n_ex = n // HW
    per_w = n // NW
    ex_per_w = n_ex // NW
    steps = per_w // T
    outer = steps // 2
    mesh = plsc.VectorSubcoreMesh(core_axis_name="c", subcore_axis_name="s")

    @functools.partial(
        pl.kernel,
        out_type=jax.ShapeDtypeStruct((n_ex, HW, D), jnp.float32),
        mesh=mesh,
        scratch_types=[
            pltpu.VMEM((T,), jnp.int32),
            pltpu.VMEM((T,), jnp.int32),
            pltpu.VMEM((NG, GCHUNK), jnp.int32),
            pltpu.VMEM((NG, GCHUNK), jnp.int32),
            pltpu.VMEM((T, D), jnp.float32),
            pltpu.VMEM((T, D), jnp.float32),
            pltpu.SemaphoreType.DMA,
            pltpu.SemaphoreType.DMA,
            pltpu.SemaphoreType.DMA,
            pltpu.SemaphoreType.DMA,
            pltpu.VMEM_SHARED((10 * HW, D), jnp.float32),
        ],
    )
    def sc_fn(fused_hbm, grid_hbm, out_hbm, g0, g1, i0, i1, r0, r1,
              sg0, sg1, ss0, ss1, fused_sh):
        wid = lax.axis_index("s") * 2 + lax.axis_index("c")
        base = wid * per_w
        lane = lax.broadcasted_iota(jnp.int32, (16,), 0)
        g_v, idx_v, rows_v = (g0, g1), (i0, i1), (r0, r1)
        sg, ss = (sg0, sg1), (ss0, ss1)

        @pl.when(lax.axis_index("s") == 0)
        def _():
            pltpu.sync_copy(fused_hbm, fused_sh)
        plsc.subcore_barrier()

        def load_idx(step, buf):
            off = pl.multiple_of(base + step * T, T)
            pltpu.sync_copy(grid_hbm.at[pl.ds(off, T)], g_v[buf])
            for m in range(T // 16):
                q = m * 16
                p = (q % HW) + lane
                p = jnp.where(p >= HW, p - HW, p)
                v = g_v[buf][pl.ds(q, 16)] * HW + p
                idx_v[buf][q // GCHUNK, pl.ds(q % GCHUNK, 16)] = v

        def fire_gather(buf):
            for j in range(NG):
                pltpu.async_copy(fused_sh.at[idx_v[buf].at[j]],
                                 rows_v[buf].at[pl.ds(j * GCHUNK, GCHUNK)],
                                 sg[buf])

        def wait_gather(buf):
            for j in range(NG):
                pltpu.make_async_copy(
                    fused_hbm.at[pl.ds(0, GCHUNK)],
                    rows_v[buf].at[pl.ds(j * GCHUNK, GCHUNK)],
                    sg[buf]).wait()

        def fire_scatter(step, buf):
            ex = base // HW + step * ROWS_PER_STEP
            for e in range(ROWS_PER_STEP):
                pltpu.async_copy(rows_v[buf].at[pl.ds(e * HW, HW)],
                                 out_hbm.at[ex + e], ss[buf])

        def wait_scatter(buf):
            for e in range(ROWS_PER_STEP):
                pltpu.make_async_copy(rows_v[buf].at[pl.ds(e * HW, HW)],
                                      out_hbm.at[0], ss[buf]).wait()

        load_idx(0, 0)
        fire_gather(0)

        def outer_body(g, carry):
            s0 = 2 * g
            load_idx(s0 + 1, 1)
            wait_gather(0)
            fire_scatter(s0, 0)
            @pl.when(g > 0)
            def _():
                wait_scatter(1)
            fire_gather(1)
            @pl.when(g < outer - 1)
            def _():
                load_idx(s0 + 2, 0)
            wait_gather(1)
            fire_scatter(s0 + 1, 1)
            wait_scatter(0)
            @pl.when(g < outer - 1)
            def _():
                fire_gather(0)
            return carry

        lax.fori_loop(0, outer, outer_body, 0)
        wait_scatter(1)

    return sc_fn(fused, grid_flat)


def kernel(grid, color_table, pos_table, W, b):
    batch, height, width = grid.shape
    fused = _make_fused(color_table, pos_table, W, b)
    flat = grid.reshape(-1).astype(jnp.int32)
    return _sc_gather(fused, flat)
